# baseline (speedup 1.0000x reference)
import functools

import jax
import jax.numpy as jnp
from jax import lax
from jax.experimental import pallas as pl
from jax.experimental.pallas import tpu as pltpu
from jax.experimental.pallas import tpu_sc as plsc

N = 10000
E = 320000
D = 128
DE = 16
G = 16

NB = 10
NBN = 5
BN = N // NBN
NP8 = N // 8
EP8 = E // 8

NC = 2
NS = 16
NW = NC * NS
EPT = E // NW
CHUNK = 1000
CP8 = CHUNK // 8
NCHUNK = EPT // CHUNK
NPAD = 10240
NH = 2
HE = E // NH
HEPT = HE // NW
HNCH = HEPT // CHUNK
RPZ = NPAD // NS

_HI = lax.Precision.HIGHEST


def _dot(a, b):
    return jnp.dot(a, b, precision=_HI)


def _ab_body(xp_ref, ohp_ref, u_ref, wesb_ref, wedb_ref, weu_ref,
             m_ref, ap_ref, bp_ref):
    ue = _dot(u_ref[...], weu_ref[...])
    uer = jnp.concatenate([ue] * 8, axis=0)
    ueb = jnp.concatenate([uer] * 8, axis=1) * m_ref[...]
    ap_ref[...] = _dot(xp_ref[...], wesb_ref[...]) + _dot(ohp_ref[...], ueb)
    bp_ref[...] = _dot(xp_ref[...], wedb_ref[...])


def _pre_ab(xp, ohp, u, wesb, wedb, weu, m):
    return pl.pallas_call(
        _ab_body,
        out_shape=[
            jax.ShapeDtypeStruct((NP8, 128), jnp.float32),
            jax.ShapeDtypeStruct((NP8, 128), jnp.float32),
        ],
    )(xp, ohp, u, wesb, wedb, weu, m)


def _c_body(eap_ref, weab_ref, bet_ref, cp_ref):
    cp_ref[...] = _dot(eap_ref[...], weab_ref[...]) + bet_ref[...]


def _pre_c(eap, weab, bet):
    HP8 = HE // 8
    BEP = HP8 // NB
    return pl.pallas_call(
        _c_body,
        grid=(NB,),
        in_specs=[
            pl.BlockSpec((BEP, 128), lambda i: (i, 0)),
            pl.BlockSpec((128, 128), lambda i: (0, 0)),
            pl.BlockSpec((1, 128), lambda i: (0, 0)),
        ],
        out_specs=pl.BlockSpec((BEP, 128), lambda i: (i, 0)),
        out_shape=jax.ShapeDtypeStruct((HE // 8, 128), jnp.float32),
    )(eap, weab, bet)


CHT = 400
NCHT = EPT // CHT


def _scin_body(ea_hbm, eaf_hbm, ebuf, obuf, sem):
    c = lax.axis_index("c")
    s = lax.axis_index("s")
    wid = s * NC + c
    base0 = wid * EPT

    def chunk_body(k, carry):
        base = base0 + k * CHT
        pltpu.sync_copy(ea_hbm.at[pl.ds(base, CHT)], ebuf)

        def row_body(r, cc):
            for j in range(8):
                i = r * 8 + j
                obuf[pl.ds(i * DE, DE)] = ebuf[i]
            return cc

        lax.fori_loop(0, CHT // 8, row_body, 0)
        pltpu.sync_copy(obuf, eaf_hbm.at[pl.ds(base * DE, CHT * DE)])
        return carry

    lax.fori_loop(0, NCHT, chunk_body, 0)


def _sc_in(ea):
    mesh = plsc.VectorSubcoreMesh(core_axis_name="c", subcore_axis_name="s",
                                  num_cores=NC, num_subcores=NS)
    return pl.kernel(
        _scin_body,
        out_type=jax.ShapeDtypeStruct((E * DE,), jnp.float32),
        mesh=mesh,
        scratch_types=[
            pltpu.VMEM((CHT, DE), jnp.float32),
            pltpu.VMEM((CHT * DE,), jnp.float32),
            pltpu.SemaphoreType.DMA,
        ],
    )(ea)


def _scout_body(e2f_hbm, e2_hbm, vbuf, obuf, sem):
    c = lax.axis_index("c")
    s = lax.axis_index("s")
    wid = s * NC + c
    base0 = wid * EPT

    def chunk_body(k, carry):
        base = base0 + k * CHT
        pltpu.sync_copy(e2f_hbm.at[pl.ds(base * DE, CHT * DE)], vbuf)

        def row_body(r, cc):
            for j in range(8):
                i = r * 8 + j
                obuf[i] = vbuf[pl.ds(i * DE, DE)]
            return cc

        lax.fori_loop(0, CHT // 8, row_body, 0)
        pltpu.sync_copy(obuf, e2_hbm.at[pl.ds(base, CHT)])
        return carry

    lax.fori_loop(0, NCHT, chunk_body, 0)


def _sc_out(e2f):
    mesh = plsc.VectorSubcoreMesh(core_axis_name="c", subcore_axis_name="s",
                                  num_cores=NC, num_subcores=NS)
    return pl.kernel(
        _scout_body,
        out_type=jax.ShapeDtypeStruct((E, DE), jnp.float32),
        mesh=mesh,
        scratch_types=[
            pltpu.VMEM((CHT * DE,), jnp.float32),
            pltpu.VMEM((CHT, DE), jnp.float32),
            pltpu.SemaphoreType.DMA,
        ],
    )(e2f)


def _sc_body(a_hbm, b_hbm, cp_hbm, src_hbm, dst_hbm, zero_hbm,
             e2_hbm, aggp_hbm,
             sidx0, didx0, arows0, brows0, sem0,
             sidx1, didx1, arows1, brows1, sem1,
             crows, o16, agg_sh):
    c = lax.axis_index("c")
    s = lax.axis_index("s")
    wid = s * NC + c

    pltpu.sync_copy(zero_hbm.at[pl.ds(s * RPZ, RPZ)],
                    agg_sh.at[pl.ds(s * RPZ, RPZ)])
    plsc.subcore_barrier()

    base0 = wid * HEPT
    slots = ((sidx0, didx0, arows0, brows0, sem0),
             (sidx1, didx1, arows1, brows1, sem1))

    def fire(cidx, sl):
        sidx_, didx_, ar_, br_, sem_ = sl
        base = base0 + cidx * CHUNK
        pltpu.sync_copy(src_hbm.at[pl.ds(base, CHUNK)], sidx_)
        pltpu.sync_copy(dst_hbm.at[pl.ds(base, CHUNK)], didx_)
        pltpu.async_copy(a_hbm.at[sidx_], ar_, sem_)
        pltpu.async_copy(b_hbm.at[didx_], br_, sem_)

    fire(0, slots[0])
    fire(1, slots[1])

    def body(k, carry):
        for b in (0, 1):
            sl = slots[b]
            sidx_, didx_, ar_, br_, sem_ = sl
            cidx = 2 * k + b

            @pl.when(cidx < HNCH)
            def _do_chunk():
                base = base0 + cidx * CHUNK
                pbase = base // 8
                pltpu.make_async_copy(a_hbm.at[sidx_], ar_, sem_).wait()
                pltpu.make_async_copy(b_hbm.at[didx_], br_, sem_).wait()
                pltpu.sync_copy(cp_hbm.at[pl.ds(pbase, CP8)], crows)

                def row_body(r, cc):
                    for j in range(8):
                        i = r * 8 + j
                        v = ar_[i] + br_[i] + crows[r, j * DE:(j + 1) * DE]
                        o16[i] = jnp.maximum(v, 0.0)
                    return cc

                lax.fori_loop(0, CP8, row_body, 0)
                pltpu.sync_copy(o16, e2_hbm.at[pl.ds(base, CHUNK)])
                pltpu.sync_copy(o16, agg_sh.at[didx_], add=True)

                @pl.when(cidx + 2 < HNCH)
                def _prefetch():
                    fire(cidx + 2, sl)
        return carry

    lax.fori_loop(0, (HNCH + 1) // 2, body, 0)
    plsc.subcore_barrier()
    pltpu.sync_copy(agg_sh.at[pl.ds(s * RPZ, RPZ)],
                    aggp_hbm.at[c, pl.ds(s * RPZ, RPZ)])


def _sc_edges(a, b, cp, src, dst, zeros):
    mesh = plsc.VectorSubcoreMesh(core_axis_name="c", subcore_axis_name="s",
                                  num_cores=NC, num_subcores=NS)
    return pl.kernel(
        _sc_body,
        out_type=(
            jax.ShapeDtypeStruct((HE, DE), jnp.float32),
            jax.ShapeDtypeStruct((NC, NPAD, DE), jnp.float32),
        ),
        mesh=mesh,
        compiler_params=pltpu.CompilerParams(use_tc_tiling_on_sc=False),
        scratch_types=[
            pltpu.VMEM((CHUNK,), jnp.int32),
            pltpu.VMEM((CHUNK,), jnp.int32),
            pltpu.VMEM((CHUNK, DE), jnp.float32),
            pltpu.VMEM((CHUNK, DE), jnp.float32),
            pltpu.SemaphoreType.DMA,
            pltpu.VMEM((CHUNK,), jnp.int32),
            pltpu.VMEM((CHUNK,), jnp.int32),
            pltpu.VMEM((CHUNK, DE), jnp.float32),
            pltpu.VMEM((CHUNK, DE), jnp.float32),
            pltpu.SemaphoreType.DMA,
            pltpu.VMEM((CP8, 128), jnp.float32),
            pltpu.VMEM((CHUNK, DE), jnp.float32),
            pltpu.VMEM_SHARED((NPAD, DE), jnp.float32),
        ],
    )(a, b, cp, src, dst, zeros)


def _node_body(x_ref, aggp_ref, oh_ref, pos_ref, u_ref, wnx_ref, wnagg_ref,
               wnu_ref, bn_ref, wgx_ref, wgpos_ref, wgu_ref, bg_ref,
               x2_ref, u2_ref, xsum, psum, csum):
    i = pl.program_id(0)

    @pl.when(i == 0)
    def _init():
        xsum[...] = jnp.zeros_like(xsum)
        psum[...] = jnp.zeros_like(psum)
        csum[...] = jnp.zeros_like(csum)

    un = _dot(u_ref[...], wnu_ref[...])
    agg = aggp_ref[0] + aggp_ref[1] + aggp_ref[2] + aggp_ref[3]
    oh = oh_ref[...]
    x2 = _dot(x_ref[...], wnx_ref[...]) + _dot(agg, wnagg_ref[...]) \
        + _dot(oh, un) + bn_ref[...]
    x2 = jnp.maximum(x2, 0.0)
    x2_ref[...] = x2

    dn = (((0,), (0,)), ((), ()))
    xsum[...] += lax.dot_general(oh, x2, dn, precision=_HI)
    psum[:, :2] += lax.dot_general(oh, pos_ref[...], dn, precision=_HI)
    csum[:, :1] += lax.dot_general(
        oh, jnp.ones((BN, 1), jnp.float32), dn, precision=_HI)

    cnt = jnp.maximum(csum[:, :1], 1.0)
    xmean = xsum[...] / cnt
    pmean = psum[:, :2] / cnt
    u2 = _dot(xmean, wgx_ref[...]) + _dot(pmean, wgpos_ref[...]) \
        + _dot(u_ref[...], wgu_ref[...]) + bg_ref[...]
    u2_ref[...] = jnp.maximum(u2, 0.0)


def _node_global(x, aggp, oh, pos, u, wnx, wnagg, wnu, bn, wgx, wgpos, wgu, bg):
    return pl.pallas_call(
        _node_body,
        grid=(NBN,),
        in_specs=[
            pl.BlockSpec((BN, D), lambda i: (i, 0)),
            pl.BlockSpec((NC * NH, BN, DE), lambda i: (0, i, 0)),
            pl.BlockSpec((BN, G), lambda i: (i, 0)),
            pl.BlockSpec((BN, 2), lambda i: (i, 0)),
            pl.BlockSpec((G, D), lambda i: (0, 0)),
            pl.BlockSpec((D, D), lambda i: (0, 0)),
            pl.BlockSpec((DE, D), lambda i: (0, 0)),
            pl.BlockSpec((D, D), lambda i: (0, 0)),
            pl.BlockSpec((1, D), lambda i: (0, 0)),
            pl.BlockSpec((D, D), lambda i: (0, 0)),
            pl.BlockSpec((2, D), lambda i: (0, 0)),
            pl.BlockSpec((D, D), lambda i: (0, 0)),
            pl.BlockSpec((1, D), lambda i: (0, 0)),
        ],
        out_specs=[
            pl.BlockSpec((BN, D), lambda i: (i, 0)),
            pl.BlockSpec((G, D), lambda i: (0, 0)),
        ],
        out_shape=[
            jax.ShapeDtypeStruct((N, D), jnp.float32),
            jax.ShapeDtypeStruct((G, D), jnp.float32),
        ],
        scratch_shapes=[
            pltpu.VMEM((G, D), jnp.float32),
            pltpu.VMEM((G, 128), jnp.float32),
            pltpu.VMEM((G, 128), jnp.float32),
        ],
    )(x, aggp, oh, pos, u, wnx, wnagg, wnu, bn, wgx, wgpos, wgu, bg)


def _blockdiag8(w):
    return jax.scipy.linalg.block_diag(*([w] * 8))


def kernel(x, edge_index, edge_attr, u, batch, polar_pos,
           W_e, b_e, W_n, b_n, W_g, b_g):
    src = edge_index[0]
    dst = edge_index[1]
    gids = jnp.arange(G, dtype=jnp.int32)
    oh = (batch[:, None] == gids[None, :]).astype(jnp.float32)
    brep = jnp.repeat(batch.reshape(NP8, 8), DE, axis=1)
    ohp = (brep == jnp.tile(gids, 8)[None, :]).astype(jnp.float32)

    wes = W_e[:D]
    wed = W_e[D:2 * D]
    wea = W_e[2 * D:2 * D + DE]
    weu = W_e[2 * D + DE:]

    xp = x.reshape(NP8, 1024)

    wesb = _blockdiag8(wes)
    wedb = _blockdiag8(wed)
    weab = _blockdiag8(wea)
    bet = jnp.tile(b_e, 8).reshape(1, 128)
    m = _blockdiag8(jnp.ones((DE, DE), jnp.float32))

    ap, bp = _pre_ab(xp, ohp, u, wesb, wedb, weu, m)
    a = ap.reshape(N, DE)
    b = bp.reshape(N, DE)

    zeros = jnp.zeros((NPAD, DE), jnp.float32)
    e2h = []
    agh = []
    for h in range(NH):
        eap_h = edge_attr[h * HE:(h + 1) * HE].reshape(HE // 8, 128)
        cp_h = _pre_c(eap_h, weab, bet)
        src_h = src[h * HE:(h + 1) * HE]
        dst_h = dst[h * HE:(h + 1) * HE]
        e2_h, aggp_h = _sc_edges(a, b, cp_h, src_h, dst_h, zeros)
        e2h.append(e2_h)
        agh.append(aggp_h)
    edge_attr2 = jnp.concatenate(e2h, axis=0)
    aggp = jnp.concatenate(agh, axis=0)

    wnx = W_n[:D]
    wnagg = W_n[D:D + DE]
    wnu = W_n[D + DE:]
    wgx = W_g[:D]
    wgpos = W_g[D:D + 2]
    wgu = W_g[D + 2:]

    x2, u2 = _node_global(x, aggp, oh, polar_pos, u, wnx, wnagg, wnu,
                          b_n.reshape(1, D), wgx, wgpos, wgu,
                          b_g.reshape(1, D))
    return (x2, edge_attr2, u2)


# speedup vs baseline: 1.2343x; 1.2343x over previous
import functools

import jax
import jax.numpy as jnp
from jax import lax
from jax.experimental import pallas as pl
from jax.experimental.pallas import tpu as pltpu
from jax.experimental.pallas import tpu_sc as plsc

N = 10000
E = 320000
D = 128
DE = 16
G = 16

NB = 10
NBN = 5
BN = N // NBN
NP8 = N // 8
EP8 = E // 8

NC = 2
NS = 16
NW = NC * NS
EPT = E // NW
CHUNK = 1000
CP8 = CHUNK // 8
NCHUNK = EPT // CHUNK
NPAD = 10240
RPZ = NPAD // NS

_HI = lax.Precision.HIGHEST


def _dot(a, b):
    return jnp.dot(a, b, precision=_HI)


def _ab_body(xp_ref, ohp_ref, u_ref, wesb_ref, wedb_ref, weu_ref,
             m_ref, ap_ref, bp_ref):
    ue = _dot(u_ref[...], weu_ref[...])
    uer = jnp.concatenate([ue] * 8, axis=0)
    ueb = jnp.concatenate([uer] * 8, axis=1) * m_ref[...]
    ap_ref[...] = _dot(xp_ref[...], wesb_ref[...]) + _dot(ohp_ref[...], ueb)
    bp_ref[...] = _dot(xp_ref[...], wedb_ref[...])


def _pre_ab(xp, ohp, u, wesb, wedb, weu, m):
    return pl.pallas_call(
        _ab_body,
        out_shape=[
            jax.ShapeDtypeStruct((NP8, 128), jnp.float32),
            jax.ShapeDtypeStruct((NP8, 128), jnp.float32),
        ],
    )(xp, ohp, u, wesb, wedb, weu, m)


def _c_body(eap_ref, weab_ref, bet_ref, cp_ref):
    cp_ref[...] = _dot(eap_ref[...], weab_ref[...]) + bet_ref[...]


def _pre_c(eap, weab, bet):
    BEP = EP8 // NB
    return pl.pallas_call(
        _c_body,
        grid=(NB,),
        in_specs=[
            pl.BlockSpec((BEP, 128), lambda i: (i, 0)),
            pl.BlockSpec((128, 128), lambda i: (0, 0)),
            pl.BlockSpec((1, 128), lambda i: (0, 0)),
        ],
        out_specs=pl.BlockSpec((BEP, 128), lambda i: (i, 0)),
        out_shape=jax.ShapeDtypeStruct((EP8, 128), jnp.float32),
    )(eap, weab, bet)


CHT = 400
NCHT = EPT // CHT


def _scin_body(ea_hbm, eaf_hbm, ebuf, obuf, sem):
    c = lax.axis_index("c")
    s = lax.axis_index("s")
    wid = s * NC + c
    base0 = wid * EPT

    def chunk_body(k, carry):
        base = base0 + k * CHT
        pltpu.sync_copy(ea_hbm.at[pl.ds(base, CHT)], ebuf)

        def row_body(r, cc):
            for j in range(8):
                i = r * 8 + j
                obuf[pl.ds(i * DE, DE)] = ebuf[i]
            return cc

        lax.fori_loop(0, CHT // 8, row_body, 0)
        pltpu.sync_copy(obuf, eaf_hbm.at[pl.ds(base * DE, CHT * DE)])
        return carry

    lax.fori_loop(0, NCHT, chunk_body, 0)


def _sc_in(ea):
    mesh = plsc.VectorSubcoreMesh(core_axis_name="c", subcore_axis_name="s",
                                  num_cores=NC, num_subcores=NS)
    return pl.kernel(
        _scin_body,
        out_type=jax.ShapeDtypeStruct((E * DE,), jnp.float32),
        mesh=mesh,
        scratch_types=[
            pltpu.VMEM((CHT, DE), jnp.float32),
            pltpu.VMEM((CHT * DE,), jnp.float32),
            pltpu.SemaphoreType.DMA,
        ],
    )(ea)


def _scout_body(e2f_hbm, e2_hbm, vbuf, obuf, sem):
    c = lax.axis_index("c")
    s = lax.axis_index("s")
    wid = s * NC + c
    base0 = wid * EPT

    def chunk_body(k, carry):
        base = base0 + k * CHT
        pltpu.sync_copy(e2f_hbm.at[pl.ds(base * DE, CHT * DE)], vbuf)

        def row_body(r, cc):
            for j in range(8):
                i = r * 8 + j
                obuf[i] = vbuf[pl.ds(i * DE, DE)]
            return cc

        lax.fori_loop(0, CHT // 8, row_body, 0)
        pltpu.sync_copy(obuf, e2_hbm.at[pl.ds(base, CHT)])
        return carry

    lax.fori_loop(0, NCHT, chunk_body, 0)


def _sc_out(e2f):
    mesh = plsc.VectorSubcoreMesh(core_axis_name="c", subcore_axis_name="s",
                                  num_cores=NC, num_subcores=NS)
    return pl.kernel(
        _scout_body,
        out_type=jax.ShapeDtypeStruct((E, DE), jnp.float32),
        mesh=mesh,
        scratch_types=[
            pltpu.VMEM((CHT * DE,), jnp.float32),
            pltpu.VMEM((CHT, DE), jnp.float32),
            pltpu.SemaphoreType.DMA,
        ],
    )(e2f)


def _sc_body(a_hbm, b_hbm, cp_hbm, src_hbm, dst_hbm, zero_hbm,
             e2_hbm, aggp_hbm,
             sidx0, didx0, arows0, brows0, sem0,
             sidx1, didx1, arows1, brows1, sem1,
             crows, o16, agg_sh):
    c = lax.axis_index("c")
    s = lax.axis_index("s")
    wid = s * NC + c

    pltpu.sync_copy(zero_hbm.at[pl.ds(s * RPZ, RPZ)],
                    agg_sh.at[pl.ds(s * RPZ, RPZ)])
    plsc.subcore_barrier()

    base0 = wid * EPT
    slots = ((sidx0, didx0, arows0, brows0, sem0),
             (sidx1, didx1, arows1, brows1, sem1))

    def fire(cidx, sl):
        sidx_, didx_, ar_, br_, sem_ = sl
        base = base0 + cidx * CHUNK
        pltpu.sync_copy(src_hbm.at[pl.ds(base, CHUNK)], sidx_)
        pltpu.sync_copy(dst_hbm.at[pl.ds(base, CHUNK)], didx_)
        pltpu.async_copy(a_hbm.at[sidx_], ar_, sem_)
        pltpu.async_copy(b_hbm.at[didx_], br_, sem_)

    fire(0, slots[0])
    fire(1, slots[1])

    def body(k, carry):
        for b in (0, 1):
            sl = slots[b]
            sidx_, didx_, ar_, br_, sem_ = sl
            cidx = 2 * k + b
            base = base0 + cidx * CHUNK
            pbase = base // 8
            pltpu.make_async_copy(a_hbm.at[sidx_], ar_, sem_).wait()
            pltpu.make_async_copy(b_hbm.at[didx_], br_, sem_).wait()
            pltpu.sync_copy(cp_hbm.at[pl.ds(pbase, CP8)], crows)

            def row_body(r, cc):
                for j in range(8):
                    i = r * 8 + j
                    v = ar_[i] + br_[i] + crows[r, j * DE:(j + 1) * DE]
                    o16[i] = jnp.maximum(v, 0.0)
                return cc

            lax.fori_loop(0, CP8, row_body, 0)
            pltpu.sync_copy(o16, e2_hbm.at[pl.ds(base, CHUNK)])
            pltpu.sync_copy(o16, agg_sh.at[didx_], add=True)

            @pl.when(cidx + 2 < NCHUNK)
            def _prefetch():
                fire(cidx + 2, sl)
        return carry

    lax.fori_loop(0, NCHUNK // 2, body, 0)
    plsc.subcore_barrier()
    pltpu.sync_copy(agg_sh.at[pl.ds(s * RPZ, RPZ)],
                    aggp_hbm.at[c, pl.ds(s * RPZ, RPZ)])


def _sc_edges(a, b, cp, src, dst, zeros):
    mesh = plsc.VectorSubcoreMesh(core_axis_name="c", subcore_axis_name="s",
                                  num_cores=NC, num_subcores=NS)
    return pl.kernel(
        _sc_body,
        out_type=(
            jax.ShapeDtypeStruct((E, DE), jnp.float32),
            jax.ShapeDtypeStruct((NC, NPAD, DE), jnp.float32),
        ),
        mesh=mesh,
        compiler_params=pltpu.CompilerParams(use_tc_tiling_on_sc=False),
        scratch_types=[
            pltpu.VMEM((CHUNK,), jnp.int32),
            pltpu.VMEM((CHUNK,), jnp.int32),
            pltpu.VMEM((CHUNK, DE), jnp.float32),
            pltpu.VMEM((CHUNK, DE), jnp.float32),
            pltpu.SemaphoreType.DMA,
            pltpu.VMEM((CHUNK,), jnp.int32),
            pltpu.VMEM((CHUNK,), jnp.int32),
            pltpu.VMEM((CHUNK, DE), jnp.float32),
            pltpu.VMEM((CHUNK, DE), jnp.float32),
            pltpu.SemaphoreType.DMA,
            pltpu.VMEM((CP8, 128), jnp.float32),
            pltpu.VMEM((CHUNK, DE), jnp.float32),
            pltpu.VMEM_SHARED((NPAD, DE), jnp.float32),
        ],
    )(a, b, cp, src, dst, zeros)


def _node_body(x_ref, aggp_ref, oh_ref, pos_ref, u_ref, wnx_ref, wnagg_ref,
               wnu_ref, bn_ref, wgx_ref, wgpos_ref, wgu_ref, bg_ref,
               x2_ref, u2_ref, xsum, psum, csum):
    i = pl.program_id(0)

    @pl.when(i == 0)
    def _init():
        xsum[...] = jnp.zeros_like(xsum)
        psum[...] = jnp.zeros_like(psum)
        csum[...] = jnp.zeros_like(csum)

    un = _dot(u_ref[...], wnu_ref[...])
    agg = aggp_ref[0] + aggp_ref[1]
    oh = oh_ref[...]
    x2 = _dot(x_ref[...], wnx_ref[...]) + _dot(agg, wnagg_ref[...]) \
        + _dot(oh, un) + bn_ref[...]
    x2 = jnp.maximum(x2, 0.0)
    x2_ref[...] = x2

    dn = (((0,), (0,)), ((), ()))
    xsum[...] += lax.dot_general(oh, x2, dn, precision=_HI)
    psum[:, :2] += lax.dot_general(oh, pos_ref[...], dn, precision=_HI)
    csum[:, :1] += lax.dot_general(
        oh, jnp.ones((BN, 1), jnp.float32), dn, precision=_HI)

    cnt = jnp.maximum(csum[:, :1], 1.0)
    xmean = xsum[...] / cnt
    pmean = psum[:, :2] / cnt
    u2 = _dot(xmean, wgx_ref[...]) + _dot(pmean, wgpos_ref[...]) \
        + _dot(u_ref[...], wgu_ref[...]) + bg_ref[...]
    u2_ref[...] = jnp.maximum(u2, 0.0)


def _node_global(x, aggp, oh, pos, u, wnx, wnagg, wnu, bn, wgx, wgpos, wgu, bg):
    return pl.pallas_call(
        _node_body,
        grid=(NBN,),
        in_specs=[
            pl.BlockSpec((BN, D), lambda i: (i, 0)),
            pl.BlockSpec((NC, BN, DE), lambda i: (0, i, 0)),
            pl.BlockSpec((BN, G), lambda i: (i, 0)),
            pl.BlockSpec((BN, 2), lambda i: (i, 0)),
            pl.BlockSpec((G, D), lambda i: (0, 0)),
            pl.BlockSpec((D, D), lambda i: (0, 0)),
            pl.BlockSpec((DE, D), lambda i: (0, 0)),
            pl.BlockSpec((D, D), lambda i: (0, 0)),
            pl.BlockSpec((1, D), lambda i: (0, 0)),
            pl.BlockSpec((D, D), lambda i: (0, 0)),
            pl.BlockSpec((2, D), lambda i: (0, 0)),
            pl.BlockSpec((D, D), lambda i: (0, 0)),
            pl.BlockSpec((1, D), lambda i: (0, 0)),
        ],
        out_specs=[
            pl.BlockSpec((BN, D), lambda i: (i, 0)),
            pl.BlockSpec((G, D), lambda i: (0, 0)),
        ],
        out_shape=[
            jax.ShapeDtypeStruct((N, D), jnp.float32),
            jax.ShapeDtypeStruct((G, D), jnp.float32),
        ],
        scratch_shapes=[
            pltpu.VMEM((G, D), jnp.float32),
            pltpu.VMEM((G, 128), jnp.float32),
            pltpu.VMEM((G, 128), jnp.float32),
        ],
    )(x, aggp, oh, pos, u, wnx, wnagg, wnu, bn, wgx, wgpos, wgu, bg)


def _blockdiag8(w):
    return jax.scipy.linalg.block_diag(*([w] * 8))


def kernel(x, edge_index, edge_attr, u, batch, polar_pos,
           W_e, b_e, W_n, b_n, W_g, b_g):
    src = edge_index[0]
    dst = edge_index[1]
    gids = jnp.arange(G, dtype=jnp.int32)
    oh = (batch[:, None] == gids[None, :]).astype(jnp.float32)
    brep = jnp.repeat(batch.reshape(NP8, 8), DE, axis=1)
    ohp = (brep == jnp.tile(gids, 8)[None, :]).astype(jnp.float32)

    wes = W_e[:D]
    wed = W_e[D:2 * D]
    wea = W_e[2 * D:2 * D + DE]
    weu = W_e[2 * D + DE:]

    xp = x.reshape(NP8, 1024)
    eap = edge_attr.reshape(EP8, 128)
    wesb = _blockdiag8(wes)
    wedb = _blockdiag8(wed)
    weab = _blockdiag8(wea)
    bet = jnp.tile(b_e, 8).reshape(1, 128)
    m = _blockdiag8(jnp.ones((DE, DE), jnp.float32))

    ap, bp = _pre_ab(xp, ohp, u, wesb, wedb, weu, m)
    cp = _pre_c(eap, weab, bet)
    a = ap.reshape(N, DE)
    b = bp.reshape(N, DE)

    zeros = jnp.zeros((NPAD, DE), jnp.float32)
    edge_attr2, aggp = _sc_edges(a, b, cp, src, dst, zeros)

    wnx = W_n[:D]
    wnagg = W_n[D:D + DE]
    wnu = W_n[D + DE:]
    wgx = W_g[:D]
    wgpos = W_g[D:D + 2]
    wgu = W_g[D + 2:]

    x2, u2 = _node_global(x, aggp, oh, polar_pos, u, wnx, wnagg, wnu,
                          b_n.reshape(1, D), wgx, wgpos, wgu,
                          b_g.reshape(1, D))
    return (x2, edge_attr2, u2)
